# Initial kernel scaffold; baseline (speedup 1.0000x reference)
import jax
import jax.numpy as jnp
from jax.experimental import pallas as pl


def kernel(x, Wr, W1, W2):
    raise NotImplementedError("write your pallas kernel here")


# speedup vs baseline: 1.0839x; 1.0839x over previous
import functools

import jax
import jax.numpy as jnp
from jax.experimental import pallas as pl
from jax.experimental.pallas import tpu as pltpu

_E = 8
_TOPK = 2


def _moe_dense_body(x_ref, wr_ref, w1_ref, w2_ref, out_ref, comb_ref, *, bm):
    e = pl.program_id(0)
    ne = pl.num_programs(0)
    t = pl.program_id(1)
    rows = pl.ds(t * bm, bm)

    @pl.when(e == 0)
    def _router():
        xl = x_ref[...]
        logits = jnp.dot(xl, wr_ref[...], preferred_element_type=jnp.float32)
        i1 = jnp.argmax(logits, axis=1)
        eye = jax.lax.broadcasted_iota(jnp.int32, logits.shape, 1)
        oh1 = (eye == i1[:, None])
        neg = jnp.finfo(jnp.float32).min
        l2 = jnp.where(oh1, neg, logits)
        i2 = jnp.argmax(l2, axis=1)
        oh2 = (eye == i2[:, None])
        m1 = jnp.max(logits, axis=1, keepdims=True)
        m2 = jnp.max(l2, axis=1, keepdims=True)
        s = jnp.exp(m2 - m1)
        g1 = 1.0 / (1.0 + s)
        g2 = s / (1.0 + s)
        comb_ref[rows, :] = jnp.where(oh1, g1, 0.0) + jnp.where(oh2, g2, 0.0)

    x = x_ref[...]
    h = jax.nn.gelu(jnp.dot(x, w1_ref[0], preferred_element_type=jnp.float32))
    o = jnp.dot(h, w2_ref[0], preferred_element_type=jnp.float32)
    comb = comb_ref[rows, :]
    sel = jax.lax.broadcasted_iota(jnp.int32, comb.shape, 1) == e
    col = jnp.sum(jnp.where(sel, comb, 0.0), axis=1, keepdims=True)
    contrib = col * o

    @pl.when(e == 0)
    def _init():
        out_ref[rows, :] = contrib

    @pl.when(e > 0)
    def _acc():
        out_ref[rows, :] += contrib


def kernel(x, Wr, W1, W2):
    B, S, D = x.shape
    T = B * S
    F = W1.shape[-1]
    xf = x.reshape(T, D)
    BM = 256
    nt = T // BM

    out = pl.pallas_call(
        functools.partial(_moe_dense_body, bm=BM),
        grid=(_E, nt),
        in_specs=[
            pl.BlockSpec((BM, D), lambda e, t: (t, 0)),
            pl.BlockSpec((D, _E), lambda e, t: (0, 0)),
            pl.BlockSpec((1, D, F), lambda e, t: (e, 0, 0)),
            pl.BlockSpec((1, F, D), lambda e, t: (e, 0, 0)),
        ],
        out_specs=pl.BlockSpec((T, D), lambda e, t: (0, 0)),
        out_shape=jax.ShapeDtypeStruct((T, D), jnp.float32),
        scratch_shapes=[
            pltpu.VMEM((T, _E), jnp.float32),
        ],
        compiler_params=pltpu.CompilerParams(
            dimension_semantics=("arbitrary", "arbitrary"),
        ),
    )(xf, Wr, W1, W2)
    return out.reshape(B, S, D)


# speedup vs baseline: 1.5320x; 1.4134x over previous
import functools

import jax
import jax.numpy as jnp
from jax import lax
from jax.experimental import pallas as pl
from jax.experimental.pallas import tpu as pltpu
from jax.experimental.pallas import tpu_sc as plsc

_E = 8
_T = 4096
_D = 1024
_F = 2048
_GM = 128
_PT = 2 * _T + _E * _GM
_NT = _PT // _GM
_NTP = ((_NT + 15) // 16) * 16
_NW = 32
_CW = _T // _NW
_BMR = 256


def _router_body(x_ref, wr_ref, e0_ref, e1_ref, r0_ref, r1_ref,
                 g0_ref, g1_ref, cnt_ref, acc_ref):
    t = pl.program_id(0)
    nt = pl.num_programs(0)

    @pl.when(t == 0)
    def _init():
        acc_ref[...] = jnp.zeros_like(acc_ref)

    x = x_ref[...]
    logits = jnp.dot(x, wr_ref[...], preferred_element_type=jnp.float32)
    i1 = jnp.argmax(logits, axis=1)
    eye = jax.lax.broadcasted_iota(jnp.int32, logits.shape, 1)
    oh1 = eye == i1[:, None]
    neg = jnp.finfo(jnp.float32).min
    l2 = jnp.where(oh1, neg, logits)
    i2 = jnp.argmax(l2, axis=1)
    oh2 = eye == i2[:, None]
    m1 = jnp.max(logits, axis=1, keepdims=True)
    m2 = jnp.max(l2, axis=1, keepdims=True)
    s = jnp.exp(m2 - m1)
    ga = 1.0 / (1.0 + s)
    g0_ref[...] = ga
    g1_ref[...] = s * ga
    e0_ref[...] = i1[:, None].astype(jnp.int32)
    e1_ref[...] = i2[:, None].astype(jnp.int32)

    s0 = oh1.astype(jnp.float32)
    s1 = oh2.astype(jnp.float32)
    row = jax.lax.broadcasted_iota(jnp.int32, (_BMR, _BMR), 0)
    col = jax.lax.broadcasted_iota(jnp.int32, (_BMR, _BMR), 1)
    tri = (row > col).astype(jnp.float32)
    c0 = jnp.dot(tri, s0, preferred_element_type=jnp.float32) + acc_ref[0:1, :]
    c1 = jnp.dot(tri, s1, preferred_element_type=jnp.float32) + acc_ref[1:2, :]
    r0_ref[...] = jnp.sum(c0 * s0, axis=1, keepdims=True).astype(jnp.int32)
    r1_ref[...] = jnp.sum(c1 * s1, axis=1, keepdims=True).astype(jnp.int32)
    acc_ref[0:1, :] += jnp.sum(s0, axis=0, keepdims=True)
    acc_ref[1:2, :] += jnp.sum(s1, axis=0, keepdims=True)

    @pl.when(t == nt - 1)
    def _fin():
        cnt_ref[...] = acc_ref[...].astype(jnp.int32)


def _router(xf, Wr):
    nt = _T // _BMR
    return pl.pallas_call(
        _router_body,
        grid=(nt,),
        in_specs=[
            pl.BlockSpec((_BMR, _D), lambda t: (t, 0)),
            pl.BlockSpec((_D, _E), lambda t: (0, 0)),
        ],
        out_specs=[
            pl.BlockSpec((_BMR, 1), lambda t: (t, 0)),
            pl.BlockSpec((_BMR, 1), lambda t: (t, 0)),
            pl.BlockSpec((_BMR, 1), lambda t: (t, 0)),
            pl.BlockSpec((_BMR, 1), lambda t: (t, 0)),
            pl.BlockSpec((_BMR, 1), lambda t: (t, 0)),
            pl.BlockSpec((_BMR, 1), lambda t: (t, 0)),
            pl.BlockSpec((2, _E), lambda t: (0, 0)),
        ],
        out_shape=[
            jax.ShapeDtypeStruct((_T, 1), jnp.int32),
            jax.ShapeDtypeStruct((_T, 1), jnp.int32),
            jax.ShapeDtypeStruct((_T, 1), jnp.int32),
            jax.ShapeDtypeStruct((_T, 1), jnp.int32),
            jax.ShapeDtypeStruct((_T, 1), jnp.float32),
            jax.ShapeDtypeStruct((_T, 1), jnp.float32),
            jax.ShapeDtypeStruct((2, _E), jnp.int32),
        ],
        scratch_shapes=[pltpu.VMEM((2, _E), jnp.float32)],
        compiler_params=pltpu.CompilerParams(
            dimension_semantics=("arbitrary",),
        ),
    )(xf, Wr)


def _dispatch_body(x_hbm, e0_hbm, e1_hbm, r0_hbm, r1_hbm, cnt_hbm,
                   xs_hbm, p0_hbm, p1_hbm, texp_hbm, act_hbm,
                   cnt_v, off_v, cnt0_v, tot_v,
                   ew_v, rw_v, pos_v, meta_v, act_v, xbuf, sem):
    wid = lax.axis_index("c") * 16 + lax.axis_index("s")
    lane = jax.lax.broadcasted_iota(jnp.int32, (16,), 0)

    pltpu.sync_copy(cnt_hbm, cnt_v)
    c0 = plsc.load_gather(cnt_v, [lane & 7])
    c1 = plsc.load_gather(cnt_v, [(lane & 7) + 8])
    tot = jnp.where(lane < 8, c0 + c1, 0)
    pad = ((tot + (_GM - 1)) >> 7) << 7
    incl = plsc.cumsum(pad)
    offex = incl - pad
    off_v[...] = offex
    cnt0_v[...] = jnp.where(lane < 8, c0, 0)
    tot_v[...] = tot

    tb = wid * _CW
    for k in range(2):
        e_hbm = e0_hbm if k == 0 else e1_hbm
        r_hbm = r0_hbm if k == 0 else r1_hbm
        p_hbm = p0_hbm if k == 0 else p1_hbm
        pltpu.sync_copy(e_hbm.at[pl.ds(tb, _CW)], ew_v)
        pltpu.sync_copy(r_hbm.at[pl.ds(tb, _CW)], rw_v)
        for h in range(2):
            for j in range(4):
                ev = ew_v[pl.ds(h * 64 + j * 16, 16)]
                rv = rw_v[pl.ds(h * 64 + j * 16, 16)]
                p = plsc.load_gather(off_v, [ev]) + rv
                if k == 1:
                    p = p + plsc.load_gather(cnt0_v, [ev])
                pos_v[h, pl.ds(j * 16, 16)] = p
            pltpu.sync_copy(pos_v.at[h], p_hbm.at[pl.ds(tb + h * 64, 64)])
            pltpu.sync_copy(x_hbm.at[pl.ds(tb + h * 64, 64)], xbuf)
            pltpu.async_copy(xbuf, xs_hbm.at[pos_v.at[h]], sem).wait()

    @pl.when(wid == 0)
    def _meta():
        for g in range(_NTP // 16):
            iv = lane + g * 16
            ivgm = iv << 7
            texp = jnp.zeros((16,), jnp.int32)
            act = jnp.zeros((16,), jnp.int32)
            for e in range(_E):
                incl_e = jnp.sum(jnp.where(lane == e, incl, 0))
                off_e = jnp.sum(jnp.where(lane == e, offex, 0))
                tot_e = jnp.sum(jnp.where(lane == e, tot, 0))
                texp += (ivgm >= incl_e).astype(jnp.int32)
                in_e = (ivgm >= off_e) & (ivgm < off_e + tot_e)
                act += in_e.astype(jnp.int32)
            meta_v[pl.ds(g * 16, 16)] = jnp.minimum(texp, 7)
            act_v[pl.ds(g * 16, 16)] = act
        pltpu.sync_copy(meta_v, texp_hbm)
        pltpu.sync_copy(act_v, act_hbm)


def _dispatch(xf, e0, e1, r0, r1, cnt16):
    mesh = plsc.VectorSubcoreMesh(core_axis_name="c", subcore_axis_name="s")
    return pl.kernel(
        _dispatch_body,
        out_type=[
            jax.ShapeDtypeStruct((_PT, _D), jnp.float32),
            jax.ShapeDtypeStruct((_T,), jnp.int32),
            jax.ShapeDtypeStruct((_T,), jnp.int32),
            jax.ShapeDtypeStruct((_NTP,), jnp.int32),
            jax.ShapeDtypeStruct((_NTP,), jnp.int32),
        ],
        mesh=mesh,
        scratch_types=[
            pltpu.VMEM((16,), jnp.int32),
            pltpu.VMEM((16,), jnp.int32),
            pltpu.VMEM((16,), jnp.int32),
            pltpu.VMEM((16,), jnp.int32),
            pltpu.VMEM((_CW,), jnp.int32),
            pltpu.VMEM((_CW,), jnp.int32),
            pltpu.VMEM((2, 64), jnp.int32),
            pltpu.VMEM((_NTP,), jnp.int32),
            pltpu.VMEM((_NTP,), jnp.int32),
            pltpu.VMEM((64, _D), jnp.float32),
            pltpu.SemaphoreType.DMA,
        ],
        compiler_params=pltpu.CompilerParams(needs_layout_passes=False),
    )(xf, e0, e1, r0, r1, cnt16)


def _ffn_body(texp_ref, act_ref, xs_ref, w1_ref, w2_ref, hs_ref):
    i = pl.program_id(0)

    @pl.when(act_ref[i] != 0)
    def _compute():
        xb = xs_ref[...]
        h = jax.nn.gelu(
            jnp.dot(xb, w1_ref[0], preferred_element_type=jnp.float32))
        hs_ref[...] = jnp.dot(h, w2_ref[0],
                              preferred_element_type=jnp.float32)


def _ffn(texp, act, xs, W1, W2):
    grid_spec = pltpu.PrefetchScalarGridSpec(
        num_scalar_prefetch=2,
        grid=(_NT,),
        in_specs=[
            pl.BlockSpec((_GM, _D), lambda i, texp, act: (i, 0)),
            pl.BlockSpec((1, _D, _F), lambda i, texp, act: (texp[i], 0, 0)),
            pl.BlockSpec((1, _F, _D), lambda i, texp, act: (texp[i], 0, 0)),
        ],
        out_specs=pl.BlockSpec((_GM, _D), lambda i, texp, act: (i, 0)),
    )
    return pl.pallas_call(
        _ffn_body,
        grid_spec=grid_spec,
        out_shape=jax.ShapeDtypeStruct((_PT, _D), jnp.float32),
        compiler_params=pltpu.CompilerParams(
            dimension_semantics=("arbitrary",),
            vmem_limit_bytes=100 * 1024 * 1024,
        ),
    )(texp, act, xs, W1, W2)


_CH = 32


def _combine_body(hs_hbm, p0_hbm, p1_hbm, g0_hbm, g1_hbm, out_hbm,
                  i0_v, i1_v, g0_v, g1_v, buf0, buf1, obuf, sem):
    wid = lax.axis_index("c") * 16 + lax.axis_index("s")
    tb = wid * _CW
    for ch in range(_CW // _CH):
        s = tb + ch * _CH
        pltpu.sync_copy(p0_hbm.at[pl.ds(s, _CH)], i0_v)
        pltpu.sync_copy(p1_hbm.at[pl.ds(s, _CH)], i1_v)
        pltpu.sync_copy(g0_hbm.at[pl.ds(s, _CH)], g0_v)
        pltpu.sync_copy(g1_hbm.at[pl.ds(s, _CH)], g1_v)
        pltpu.async_copy(hs_hbm.at[i0_v], buf0, sem).wait()
        pltpu.async_copy(hs_hbm.at[i1_v], buf1, sem).wait()

        def row_body(r, carry):
            idx = jax.lax.broadcasted_iota(jnp.int32, (16,), 0) * 0 + r
            ga = plsc.load_gather(g0_v, [idx])
            gb = plsc.load_gather(g1_v, [idx])
            for j in range(_D // 16):
                sl = pl.ds(j * 16, 16)
                obuf[r, sl] = ga * buf0[r, sl] + gb * buf1[r, sl]
            return carry

        lax.fori_loop(0, _CH, row_body, 0)
        pltpu.sync_copy(obuf, out_hbm.at[pl.ds(s, _CH)])


def _combine(hs, p0, p1, g0, g1):
    mesh = plsc.VectorSubcoreMesh(core_axis_name="c", subcore_axis_name="s")
    return pl.kernel(
        _combine_body,
        out_type=jax.ShapeDtypeStruct((_T, _D), jnp.float32),
        mesh=mesh,
        scratch_types=[
            pltpu.VMEM((_CH,), jnp.int32),
            pltpu.VMEM((_CH,), jnp.int32),
            pltpu.VMEM((_CH,), jnp.float32),
            pltpu.VMEM((_CH,), jnp.float32),
            pltpu.VMEM((_CH, _D), jnp.float32),
            pltpu.VMEM((_CH, _D), jnp.float32),
            pltpu.VMEM((_CH, _D), jnp.float32),
            pltpu.SemaphoreType.DMA,
        ],
        compiler_params=pltpu.CompilerParams(needs_layout_passes=False),
    )(hs, p0, p1, g0, g1)


def kernel(x, Wr, W1, W2):
    B, S, D = x.shape
    xf = x.reshape(_T, D)
    e0, e1, r0, r1, g0, g1, cnt = _router(xf, Wr)
    e0 = e0.reshape(_T)
    e1 = e1.reshape(_T)
    r0 = r0.reshape(_T)
    r1 = r1.reshape(_T)
    g0 = g0.reshape(_T)
    g1 = g1.reshape(_T)
    cnt16 = cnt.reshape(2 * _E)
    xs, p0, p1, texp, act = _dispatch(xf, e0, e1, r0, r1, cnt16)
    hs = _ffn(texp, act, xs, W1, W2)
    out = _combine(hs, p0, p1, g0, g1)
    return out.reshape(B, S, D)


# speedup vs baseline: 1.5850x; 1.0346x over previous
import functools

import jax
import jax.numpy as jnp
from jax import lax
from jax.experimental import pallas as pl
from jax.experimental.pallas import tpu as pltpu
from jax.experimental.pallas import tpu_sc as plsc

_E = 8
_T = 4096
_D = 1024
_F = 2048
_GM = 128
_PT = 2 * _T + _E * _GM
_NT = _PT // _GM
_NTP = ((_NT + 15) // 16) * 16
_NW = 32
_CW = _T // _NW
_BMR = 1024


def _router_body(x_ref, wr_ref, e0_ref, e1_ref, r0_ref, r1_ref,
                 g0_ref, g1_ref, cnt_ref, acc_ref):
    t = pl.program_id(0)
    nt = pl.num_programs(0)

    @pl.when(t == 0)
    def _init():
        acc_ref[...] = jnp.zeros_like(acc_ref)

    x = x_ref[...]
    logits = jnp.dot(x, wr_ref[...], preferred_element_type=jnp.float32)
    i1 = jnp.argmax(logits, axis=1)
    eye = jax.lax.broadcasted_iota(jnp.int32, logits.shape, 1)
    oh1 = eye == i1[:, None]
    neg = jnp.finfo(jnp.float32).min
    l2 = jnp.where(oh1, neg, logits)
    i2 = jnp.argmax(l2, axis=1)
    oh2 = eye == i2[:, None]
    m1 = jnp.max(logits, axis=1, keepdims=True)
    m2 = jnp.max(l2, axis=1, keepdims=True)
    s = jnp.exp(m2 - m1)
    ga = 1.0 / (1.0 + s)
    g0_ref[...] = ga
    g1_ref[...] = s * ga
    e0_ref[...] = i1[:, None].astype(jnp.int32)
    e1_ref[...] = i2[:, None].astype(jnp.int32)

    s0 = oh1.astype(jnp.float32)
    s1 = oh2.astype(jnp.float32)
    row = jax.lax.broadcasted_iota(jnp.int32, (_BMR, _BMR), 0)
    col = jax.lax.broadcasted_iota(jnp.int32, (_BMR, _BMR), 1)
    tri = (row > col).astype(jnp.float32)
    c0 = jnp.dot(tri, s0, preferred_element_type=jnp.float32) + acc_ref[0:1, :]
    c1 = jnp.dot(tri, s1, preferred_element_type=jnp.float32) + acc_ref[1:2, :]
    r0_ref[...] = jnp.sum(c0 * s0, axis=1, keepdims=True).astype(jnp.int32)
    r1_ref[...] = jnp.sum(c1 * s1, axis=1, keepdims=True).astype(jnp.int32)
    acc_ref[0:1, :] += jnp.sum(s0, axis=0, keepdims=True)
    acc_ref[1:2, :] += jnp.sum(s1, axis=0, keepdims=True)

    @pl.when(t == nt - 1)
    def _fin():
        cnt_ref[...] = acc_ref[...].astype(jnp.int32)


def _router(xf, Wr):
    nt = _T // _BMR
    return pl.pallas_call(
        _router_body,
        grid=(nt,),
        in_specs=[
            pl.BlockSpec((_BMR, _D), lambda t: (t, 0)),
            pl.BlockSpec((_D, _E), lambda t: (0, 0)),
        ],
        out_specs=[
            pl.BlockSpec((_BMR, 1), lambda t: (t, 0)),
            pl.BlockSpec((_BMR, 1), lambda t: (t, 0)),
            pl.BlockSpec((_BMR, 1), lambda t: (t, 0)),
            pl.BlockSpec((_BMR, 1), lambda t: (t, 0)),
            pl.BlockSpec((_BMR, 1), lambda t: (t, 0)),
            pl.BlockSpec((_BMR, 1), lambda t: (t, 0)),
            pl.BlockSpec((2, _E), lambda t: (0, 0)),
        ],
        out_shape=[
            jax.ShapeDtypeStruct((_T, 1), jnp.int32),
            jax.ShapeDtypeStruct((_T, 1), jnp.int32),
            jax.ShapeDtypeStruct((_T, 1), jnp.int32),
            jax.ShapeDtypeStruct((_T, 1), jnp.int32),
            jax.ShapeDtypeStruct((_T, 1), jnp.float32),
            jax.ShapeDtypeStruct((_T, 1), jnp.float32),
            jax.ShapeDtypeStruct((2, _E), jnp.int32),
        ],
        scratch_shapes=[pltpu.VMEM((2, _E), jnp.float32)],
        compiler_params=pltpu.CompilerParams(
            dimension_semantics=("arbitrary",),
        ),
    )(xf, Wr)


def _dispatch_body(x_hbm, e0_hbm, e1_hbm, r0_hbm, r1_hbm, cnt_hbm,
                   xs_hbm, p0_hbm, p1_hbm, texp_hbm, act_hbm,
                   cnt_v, off_v, cnt0_v, tot_v,
                   ew_v, rw_v, pos_v, meta_v, act_v, xbuf, sem):
    wid = lax.axis_index("c") * 16 + lax.axis_index("s")
    lane = jax.lax.broadcasted_iota(jnp.int32, (16,), 0)

    pltpu.sync_copy(cnt_hbm, cnt_v)
    c0 = plsc.load_gather(cnt_v, [lane & 7])
    c1 = plsc.load_gather(cnt_v, [(lane & 7) + 8])
    tot = jnp.where(lane < 8, c0 + c1, 0)
    pad = ((tot + (_GM - 1)) >> 7) << 7
    incl = plsc.cumsum(pad)
    offex = incl - pad
    off_v[...] = offex
    cnt0_v[...] = jnp.where(lane < 8, c0, 0)
    tot_v[...] = tot

    tb = wid * _CW
    for k in range(2):
        e_hbm = e0_hbm if k == 0 else e1_hbm
        r_hbm = r0_hbm if k == 0 else r1_hbm
        p_hbm = p0_hbm if k == 0 else p1_hbm
        pltpu.sync_copy(e_hbm.at[pl.ds(tb, _CW)], ew_v)
        pltpu.sync_copy(r_hbm.at[pl.ds(tb, _CW)], rw_v)
        for h in range(4):
            for j in range(2):
                ev = ew_v[pl.ds(h * 32 + j * 16, 16)]
                rv = rw_v[pl.ds(h * 32 + j * 16, 16)]
                p = plsc.load_gather(off_v, [ev]) + rv
                if k == 1:
                    p = p + plsc.load_gather(cnt0_v, [ev])
                pos_v[k * 4 + h, pl.ds(j * 16, 16)] = p
            pltpu.sync_copy(pos_v.at[k * 4 + h],
                            p_hbm.at[pl.ds(tb + h * 32, 32)])
    pend = [None, None]
    for c in range(8):
        par = c % 2
        if pend[par] is not None:
            pend[par].wait()
        tok = tb + (c % 4) * 32
        pltpu.sync_copy(x_hbm.at[pl.ds(tok, 32)], xbuf.at[par])
        pend[par] = pltpu.async_copy(xbuf.at[par], xs_hbm.at[pos_v.at[c]],
                                     sem.at[par])
    pend[0].wait()
    pend[1].wait()

    @pl.when(wid == 0)
    def _meta():
        for g in range(_NTP // 16):
            iv = lane + g * 16
            ivgm = iv << 7
            texp = jnp.zeros((16,), jnp.int32)
            act = jnp.zeros((16,), jnp.int32)
            for e in range(_E):
                incl_e = jnp.sum(jnp.where(lane == e, incl, 0))
                off_e = jnp.sum(jnp.where(lane == e, offex, 0))
                tot_e = jnp.sum(jnp.where(lane == e, tot, 0))
                texp += (ivgm >= incl_e).astype(jnp.int32)
                in_e = (ivgm >= off_e) & (ivgm < off_e + tot_e)
                act += in_e.astype(jnp.int32)
            meta_v[pl.ds(g * 16, 16)] = jnp.minimum(texp, 7)
            act_v[pl.ds(g * 16, 16)] = act
        pltpu.sync_copy(meta_v, texp_hbm)
        pltpu.sync_copy(act_v, act_hbm)


def _dispatch(xf, e0, e1, r0, r1, cnt16):
    mesh = plsc.VectorSubcoreMesh(core_axis_name="c", subcore_axis_name="s")
    return pl.kernel(
        _dispatch_body,
        out_type=[
            jax.ShapeDtypeStruct((_PT, _D), jnp.float32),
            jax.ShapeDtypeStruct((_T,), jnp.int32),
            jax.ShapeDtypeStruct((_T,), jnp.int32),
            jax.ShapeDtypeStruct((_NTP,), jnp.int32),
            jax.ShapeDtypeStruct((_NTP,), jnp.int32),
        ],
        mesh=mesh,
        scratch_types=[
            pltpu.VMEM((16,), jnp.int32),
            pltpu.VMEM((16,), jnp.int32),
            pltpu.VMEM((16,), jnp.int32),
            pltpu.VMEM((16,), jnp.int32),
            pltpu.VMEM((_CW,), jnp.int32),
            pltpu.VMEM((_CW,), jnp.int32),
            pltpu.VMEM((8, 32), jnp.int32),
            pltpu.VMEM((_NTP,), jnp.int32),
            pltpu.VMEM((_NTP,), jnp.int32),
            pltpu.VMEM((2, 32, _D), jnp.float32),
            pltpu.SemaphoreType.DMA((2,)),
        ],
        compiler_params=pltpu.CompilerParams(needs_layout_passes=False),
    )(xf, e0, e1, r0, r1, cnt16)


def _ffn_body(texp_ref, act_ref, xs_ref, w1_ref, w2_ref, hs_ref):
    i = pl.program_id(0)

    @pl.when(act_ref[i] != 0)
    def _compute():
        xb = xs_ref[...]
        h = jax.nn.gelu(
            jnp.dot(xb, w1_ref[0], preferred_element_type=jnp.float32))
        hs_ref[...] = jnp.dot(h, w2_ref[0],
                              preferred_element_type=jnp.float32)


def _ffn(texp, act, xs, W1, W2):
    grid_spec = pltpu.PrefetchScalarGridSpec(
        num_scalar_prefetch=2,
        grid=(_NT,),
        in_specs=[
            pl.BlockSpec((_GM, _D), lambda i, texp, act: (i, 0)),
            pl.BlockSpec((1, _D, _F), lambda i, texp, act: (texp[i], 0, 0)),
            pl.BlockSpec((1, _F, _D), lambda i, texp, act: (texp[i], 0, 0)),
        ],
        out_specs=pl.BlockSpec((_GM, _D), lambda i, texp, act: (i, 0)),
    )
    return pl.pallas_call(
        _ffn_body,
        grid_spec=grid_spec,
        out_shape=jax.ShapeDtypeStruct((_PT, _D), jnp.float32),
        compiler_params=pltpu.CompilerParams(
            dimension_semantics=("arbitrary",),
            vmem_limit_bytes=100 * 1024 * 1024,
        ),
    )(texp, act, xs, W1, W2)


_CH = 16
_NCH = _CW // _CH


def _combine_body(hs_hbm, p0_hbm, p1_hbm, g0_hbm, g1_hbm, out_hbm,
                  i0_v, i1_v, g0_v, g1_v, buf0, buf1, obuf,
                  sem0, sem1, semo):
    wid = lax.axis_index("c") * 16 + lax.axis_index("s")
    tb = wid * _CW

    def issue(c, par):
        s = tb + c * _CH
        pltpu.sync_copy(p0_hbm.at[pl.ds(s, _CH)], i0_v.at[par])
        pltpu.sync_copy(p1_hbm.at[pl.ds(s, _CH)], i1_v.at[par])
        pltpu.sync_copy(g0_hbm.at[pl.ds(s, _CH)], g0_v.at[par])
        pltpu.sync_copy(g1_hbm.at[pl.ds(s, _CH)], g1_v.at[par])
        d0 = pltpu.async_copy(hs_hbm.at[i0_v.at[par]], buf0.at[par],
                              sem0.at[par])
        d1 = pltpu.async_copy(hs_hbm.at[i1_v.at[par]], buf1.at[par],
                              sem1.at[par])
        return d0, d1

    pend = [None, None]
    out_pend = [None, None]
    pend[0] = issue(0, 0)
    for c in range(_NCH):
        par = c % 2
        if c + 1 < _NCH:
            pend[1 - par] = issue(c + 1, 1 - par)
        pend[par][0].wait()
        pend[par][1].wait()
        if out_pend[par] is not None:
            out_pend[par].wait()

        def row_body(r, carry):
            idx = jax.lax.broadcasted_iota(jnp.int32, (16,), 0) * 0 + r
            ga = plsc.load_gather(g0_v.at[par], [idx])
            gb = plsc.load_gather(g1_v.at[par], [idx])
            for j in range(_D // 16):
                sl = pl.ds(j * 16, 16)
                obuf[par, r, sl] = (ga * buf0[par, r, sl]
                                    + gb * buf1[par, r, sl])
            return carry

        lax.fori_loop(0, _CH, row_body, 0)
        out_pend[par] = pltpu.async_copy(
            obuf.at[par], out_hbm.at[pl.ds(tb + c * _CH, _CH)], semo.at[par])
    for par in range(2):
        if out_pend[par] is not None:
            out_pend[par].wait()


def _combine(hs, p0, p1, g0, g1):
    mesh = plsc.VectorSubcoreMesh(core_axis_name="c", subcore_axis_name="s")
    return pl.kernel(
        _combine_body,
        out_type=jax.ShapeDtypeStruct((_T, _D), jnp.float32),
        mesh=mesh,
        scratch_types=[
            pltpu.VMEM((2, _CH), jnp.int32),
            pltpu.VMEM((2, _CH), jnp.int32),
            pltpu.VMEM((2, _CH), jnp.float32),
            pltpu.VMEM((2, _CH), jnp.float32),
            pltpu.VMEM((2, _CH, _D), jnp.float32),
            pltpu.VMEM((2, _CH, _D), jnp.float32),
            pltpu.VMEM((2, _CH, _D), jnp.float32),
            pltpu.SemaphoreType.DMA((2,)),
            pltpu.SemaphoreType.DMA((2,)),
            pltpu.SemaphoreType.DMA((2,)),
        ],
        compiler_params=pltpu.CompilerParams(needs_layout_passes=False),
    )(hs, p0, p1, g0, g1)


def kernel(x, Wr, W1, W2):
    B, S, D = x.shape
    xf = x.reshape(_T, D)
    e0, e1, r0, r1, g0, g1, cnt = _router(xf, Wr)
    e0 = e0.reshape(_T)
    e1 = e1.reshape(_T)
    r0 = r0.reshape(_T)
    r1 = r1.reshape(_T)
    g0 = g0.reshape(_T)
    g1 = g1.reshape(_T)
    cnt16 = cnt.reshape(2 * _E)
    xs, p0, p1, texp, act = _dispatch(xf, e0, e1, r0, r1, cnt16)
    hs = _ffn(texp, act, xs, W1, W2)
    out = _combine(hs, p0, p1, g0, g1)
    return out.reshape(B, S, D)


# speedup vs baseline: 1.6841x; 1.0626x over previous
import functools

import jax
import jax.numpy as jnp
from jax import lax
from jax.experimental import pallas as pl
from jax.experimental.pallas import tpu as pltpu
from jax.experimental.pallas import tpu_sc as plsc

_E = 8
_T = 4096
_D = 1024
_F = 2048
_GM = 256
_GMSH = _GM.bit_length() - 1
_PT = 2 * _T + _E * _GM
_NT = _PT // _GM
_NTP = ((_NT + 15) // 16) * 16
_NW = 32
_CW = _T // _NW
_BMR = 1024


def _router_body(x_ref, wr_ref, e0_ref, e1_ref, r0_ref, r1_ref,
                 g0_ref, g1_ref, cnt_ref, acc_ref):
    t = pl.program_id(0)
    nt = pl.num_programs(0)

    @pl.when(t == 0)
    def _init():
        acc_ref[...] = jnp.zeros_like(acc_ref)

    x = x_ref[...]
    logits = jnp.dot(x, wr_ref[...], preferred_element_type=jnp.float32)
    i1 = jnp.argmax(logits, axis=1)
    eye = jax.lax.broadcasted_iota(jnp.int32, logits.shape, 1)
    oh1 = eye == i1[:, None]
    neg = jnp.finfo(jnp.float32).min
    l2 = jnp.where(oh1, neg, logits)
    i2 = jnp.argmax(l2, axis=1)
    oh2 = eye == i2[:, None]
    m1 = jnp.max(logits, axis=1, keepdims=True)
    m2 = jnp.max(l2, axis=1, keepdims=True)
    s = jnp.exp(m2 - m1)
    ga = 1.0 / (1.0 + s)
    g0_ref[...] = ga
    g1_ref[...] = s * ga
    e0_ref[...] = i1[:, None].astype(jnp.int32)
    e1_ref[...] = i2[:, None].astype(jnp.int32)

    s0 = oh1.astype(jnp.float32)
    s1 = oh2.astype(jnp.float32)
    row = jax.lax.broadcasted_iota(jnp.int32, (_BMR, _BMR), 0)
    col = jax.lax.broadcasted_iota(jnp.int32, (_BMR, _BMR), 1)
    tri = (row > col).astype(jnp.float32)
    c0 = jnp.dot(tri, s0, preferred_element_type=jnp.float32) + acc_ref[0:1, :]
    c1 = jnp.dot(tri, s1, preferred_element_type=jnp.float32) + acc_ref[1:2, :]
    r0_ref[...] = jnp.sum(c0 * s0, axis=1, keepdims=True).astype(jnp.int32)
    r1_ref[...] = jnp.sum(c1 * s1, axis=1, keepdims=True).astype(jnp.int32)
    acc_ref[0:1, :] += jnp.sum(s0, axis=0, keepdims=True)
    acc_ref[1:2, :] += jnp.sum(s1, axis=0, keepdims=True)

    @pl.when(t == nt - 1)
    def _fin():
        cnt_ref[...] = acc_ref[...].astype(jnp.int32)


def _router(xf, Wr):
    nt = _T // _BMR
    return pl.pallas_call(
        _router_body,
        grid=(nt,),
        in_specs=[
            pl.BlockSpec((_BMR, _D), lambda t: (t, 0)),
            pl.BlockSpec((_D, _E), lambda t: (0, 0)),
        ],
        out_specs=[
            pl.BlockSpec((_BMR, 1), lambda t: (t, 0)),
            pl.BlockSpec((_BMR, 1), lambda t: (t, 0)),
            pl.BlockSpec((_BMR, 1), lambda t: (t, 0)),
            pl.BlockSpec((_BMR, 1), lambda t: (t, 0)),
            pl.BlockSpec((_BMR, 1), lambda t: (t, 0)),
            pl.BlockSpec((_BMR, 1), lambda t: (t, 0)),
            pl.BlockSpec((2, _E), lambda t: (0, 0)),
        ],
        out_shape=[
            jax.ShapeDtypeStruct((_T, 1), jnp.int32),
            jax.ShapeDtypeStruct((_T, 1), jnp.int32),
            jax.ShapeDtypeStruct((_T, 1), jnp.int32),
            jax.ShapeDtypeStruct((_T, 1), jnp.int32),
            jax.ShapeDtypeStruct((_T, 1), jnp.float32),
            jax.ShapeDtypeStruct((_T, 1), jnp.float32),
            jax.ShapeDtypeStruct((2, _E), jnp.int32),
        ],
        scratch_shapes=[pltpu.VMEM((2, _E), jnp.float32)],
        compiler_params=pltpu.CompilerParams(
            dimension_semantics=("arbitrary",),
        ),
    )(xf, Wr)


def _dispatch_body(x_hbm, e0_hbm, e1_hbm, r0_hbm, r1_hbm, cnt_hbm,
                   xs_hbm, p0_hbm, p1_hbm, texp_hbm, act_hbm,
                   cnt_v, off_v, cnt0_v, tot_v,
                   ew_v, rw_v, pos_v, meta_v, act_v, xbuf, sem):
    wid = lax.axis_index("c") * 16 + lax.axis_index("s")
    lane = jax.lax.broadcasted_iota(jnp.int32, (16,), 0)

    pltpu.sync_copy(cnt_hbm, cnt_v)
    c0 = plsc.load_gather(cnt_v, [lane & 7])
    c1 = plsc.load_gather(cnt_v, [(lane & 7) + 8])
    tot = jnp.where(lane < 8, c0 + c1, 0)
    pad = ((tot + (_GM - 1)) >> _GMSH) << _GMSH
    incl = plsc.cumsum(pad)
    offex = incl - pad
    off_v[...] = offex
    cnt0_v[...] = jnp.where(lane < 8, c0, 0)
    tot_v[...] = tot

    tb = wid * _CW
    for k in range(2):
        e_hbm = e0_hbm if k == 0 else e1_hbm
        r_hbm = r0_hbm if k == 0 else r1_hbm
        p_hbm = p0_hbm if k == 0 else p1_hbm
        pltpu.sync_copy(e_hbm.at[pl.ds(tb, _CW)], ew_v)
        pltpu.sync_copy(r_hbm.at[pl.ds(tb, _CW)], rw_v)
        for h in range(4):
            for j in range(2):
                ev = ew_v[pl.ds(h * 32 + j * 16, 16)]
                rv = rw_v[pl.ds(h * 32 + j * 16, 16)]
                p = plsc.load_gather(off_v, [ev]) + rv
                if k == 1:
                    p = p + plsc.load_gather(cnt0_v, [ev])
                pos_v[k * 4 + h, pl.ds(j * 16, 16)] = p
            pltpu.sync_copy(pos_v.at[k * 4 + h],
                            p_hbm.at[pl.ds(tb + h * 32, 32)])
    pend = [None, None]
    for c in range(8):
        par = c % 2
        if pend[par] is not None:
            pend[par].wait()
        tok = tb + (c % 4) * 32
        pltpu.sync_copy(x_hbm.at[pl.ds(tok, 32)], xbuf.at[par])
        pend[par] = pltpu.async_copy(xbuf.at[par], xs_hbm.at[pos_v.at[c]],
                                     sem.at[par])
    pend[0].wait()
    pend[1].wait()

    @pl.when(wid == 0)
    def _meta():
        for g in range(_NTP // 16):
            iv = lane + g * 16
            ivgm = iv << _GMSH
            texp = jnp.zeros((16,), jnp.int32)
            act = jnp.zeros((16,), jnp.int32)
            for e in range(_E):
                incl_e = jnp.sum(jnp.where(lane == e, incl, 0))
                off_e = jnp.sum(jnp.where(lane == e, offex, 0))
                tot_e = jnp.sum(jnp.where(lane == e, tot, 0))
                texp += (ivgm >= incl_e).astype(jnp.int32)
                in_e = (ivgm >= off_e) & (ivgm < off_e + tot_e)
                act += in_e.astype(jnp.int32)
            meta_v[pl.ds(g * 16, 16)] = jnp.minimum(texp, 7)
            act_v[pl.ds(g * 16, 16)] = act
        pltpu.sync_copy(meta_v, texp_hbm)
        pltpu.sync_copy(act_v, act_hbm)


def _dispatch(xf, e0, e1, r0, r1, cnt16):
    mesh = plsc.VectorSubcoreMesh(core_axis_name="c", subcore_axis_name="s")
    return pl.kernel(
        _dispatch_body,
        out_type=[
            jax.ShapeDtypeStruct((_PT, _D), jnp.float32),
            jax.ShapeDtypeStruct((_T,), jnp.int32),
            jax.ShapeDtypeStruct((_T,), jnp.int32),
            jax.ShapeDtypeStruct((_NTP,), jnp.int32),
            jax.ShapeDtypeStruct((_NTP,), jnp.int32),
        ],
        mesh=mesh,
        scratch_types=[
            pltpu.VMEM((16,), jnp.int32),
            pltpu.VMEM((16,), jnp.int32),
            pltpu.VMEM((16,), jnp.int32),
            pltpu.VMEM((16,), jnp.int32),
            pltpu.VMEM((_CW,), jnp.int32),
            pltpu.VMEM((_CW,), jnp.int32),
            pltpu.VMEM((8, 32), jnp.int32),
            pltpu.VMEM((_NTP,), jnp.int32),
            pltpu.VMEM((_NTP,), jnp.int32),
            pltpu.VMEM((2, 32, _D), jnp.float32),
            pltpu.SemaphoreType.DMA((2,)),
        ],
        compiler_params=pltpu.CompilerParams(needs_layout_passes=False),
    )(xf, e0, e1, r0, r1, cnt16)


def _ffn_body(texp_ref, act_ref, xs_ref, w1_ref, w2_ref, hs_ref):
    i = pl.program_id(0)

    @pl.when(act_ref[i] != 0)
    def _compute():
        xb = xs_ref[...]
        h = jax.nn.gelu(
            jnp.dot(xb, w1_ref[0], preferred_element_type=jnp.float32))
        hs_ref[...] = jnp.dot(h, w2_ref[0],
                              preferred_element_type=jnp.float32)


def _ffn(texp, act, xs, W1, W2):
    grid_spec = pltpu.PrefetchScalarGridSpec(
        num_scalar_prefetch=2,
        grid=(_NT,),
        in_specs=[
            pl.BlockSpec((_GM, _D), lambda i, texp, act: (i, 0)),
            pl.BlockSpec((1, _D, _F), lambda i, texp, act: (texp[i], 0, 0)),
            pl.BlockSpec((1, _F, _D), lambda i, texp, act: (texp[i], 0, 0)),
        ],
        out_specs=pl.BlockSpec((_GM, _D), lambda i, texp, act: (i, 0)),
    )
    return pl.pallas_call(
        _ffn_body,
        grid_spec=grid_spec,
        out_shape=jax.ShapeDtypeStruct((_PT, _D), jnp.float32),
        compiler_params=pltpu.CompilerParams(
            dimension_semantics=("arbitrary",),
            vmem_limit_bytes=100 * 1024 * 1024,
        ),
    )(texp, act, xs, W1, W2)


_CH = 16
_NCH = _CW // _CH


def _combine_body(hs_hbm, p0_hbm, p1_hbm, g0_hbm, g1_hbm, out_hbm,
                  i0_v, i1_v, g0_v, g1_v, buf0, buf1, obuf,
                  sem0, sem1, semo):
    wid = lax.axis_index("c") * 16 + lax.axis_index("s")
    tb = wid * _CW

    def issue(c, par):
        s = tb + c * _CH
        pltpu.sync_copy(p0_hbm.at[pl.ds(s, _CH)], i0_v.at[par])
        pltpu.sync_copy(p1_hbm.at[pl.ds(s, _CH)], i1_v.at[par])
        pltpu.sync_copy(g0_hbm.at[pl.ds(s, _CH)], g0_v.at[par])
        pltpu.sync_copy(g1_hbm.at[pl.ds(s, _CH)], g1_v.at[par])
        d0 = pltpu.async_copy(hs_hbm.at[i0_v.at[par]], buf0.at[par],
                              sem0.at[par])
        d1 = pltpu.async_copy(hs_hbm.at[i1_v.at[par]], buf1.at[par],
                              sem1.at[par])
        return d0, d1

    pend = [None, None]
    out_pend = [None, None]
    pend[0] = issue(0, 0)
    for c in range(_NCH):
        par = c % 2
        if c + 1 < _NCH:
            pend[1 - par] = issue(c + 1, 1 - par)
        pend[par][0].wait()
        pend[par][1].wait()
        if out_pend[par] is not None:
            out_pend[par].wait()

        def row_body(r, carry):
            idx = jax.lax.broadcasted_iota(jnp.int32, (16,), 0) * 0 + r
            ga = plsc.load_gather(g0_v.at[par], [idx])
            gb = plsc.load_gather(g1_v.at[par], [idx])
            for j in range(_D // 16):
                sl = pl.ds(j * 16, 16)
                obuf[par, r, sl] = (ga * buf0[par, r, sl]
                                    + gb * buf1[par, r, sl])
            return carry

        lax.fori_loop(0, _CH, row_body, 0)
        out_pend[par] = pltpu.async_copy(
            obuf.at[par], out_hbm.at[pl.ds(tb + c * _CH, _CH)], semo.at[par])
    for par in range(2):
        if out_pend[par] is not None:
            out_pend[par].wait()


def _combine(hs, p0, p1, g0, g1):
    mesh = plsc.VectorSubcoreMesh(core_axis_name="c", subcore_axis_name="s")
    return pl.kernel(
        _combine_body,
        out_type=jax.ShapeDtypeStruct((_T, _D), jnp.float32),
        mesh=mesh,
        scratch_types=[
            pltpu.VMEM((2, _CH), jnp.int32),
            pltpu.VMEM((2, _CH), jnp.int32),
            pltpu.VMEM((2, _CH), jnp.float32),
            pltpu.VMEM((2, _CH), jnp.float32),
            pltpu.VMEM((2, _CH, _D), jnp.float32),
            pltpu.VMEM((2, _CH, _D), jnp.float32),
            pltpu.VMEM((2, _CH, _D), jnp.float32),
            pltpu.SemaphoreType.DMA((2,)),
            pltpu.SemaphoreType.DMA((2,)),
            pltpu.SemaphoreType.DMA((2,)),
        ],
        compiler_params=pltpu.CompilerParams(needs_layout_passes=False),
    )(hs, p0, p1, g0, g1)


def kernel(x, Wr, W1, W2):
    B, S, D = x.shape
    xf = x.reshape(_T, D)
    e0, e1, r0, r1, g0, g1, cnt = _router(xf, Wr)
    e0 = e0.reshape(_T)
    e1 = e1.reshape(_T)
    r0 = r0.reshape(_T)
    r1 = r1.reshape(_T)
    g0 = g0.reshape(_T)
    g1 = g1.reshape(_T)
    cnt16 = cnt.reshape(2 * _E)
    xs, p0, p1, texp, act = _dispatch(xf, e0, e1, r0, r1, cnt16)
    hs = _ffn(texp, act, xs, W1, W2)
    out = _combine(hs, p0, p1, g0, g1)
    return out.reshape(B, S, D)
